# Initial kernel scaffold; baseline (speedup 1.0000x reference)
import jax
import jax.numpy as jnp
from jax.experimental import pallas as pl


def kernel(x, pos, batch, W1, b1, W2, b2, W3, b3):
    raise NotImplementedError("write your pallas kernel here")


# speedup vs baseline: 2.9405x; 2.9405x over previous
import functools

import jax
import jax.numpy as jnp
from jax.experimental import pallas as pl
from jax.experimental.pallas import tpu as pltpu

_R2 = 0.2 * 0.2
_K = 64


def _fps_body(n, m, px_ref, py_ref, pz_ref, sx_ref, sy_ref, sz_ref,
              idx_ref, qx_ref, qy_ref, qz_ref, dists_ref, flat_ref):
    shp = (px_ref.shape[0], px_ref.shape[1])
    flat = (jax.lax.broadcasted_iota(jnp.int32, shp, 0) * shp[1]
            + jax.lax.broadcasted_iota(jnp.int32, shp, 1))
    flat_ref[...] = flat.astype(jnp.float32)
    dists_ref[...] = jnp.where(flat < n, jnp.inf, -1.0)

    def body(i, carry):
        d = dists_ref[...]
        colmax = jnp.max(d, axis=0, keepdims=True)
        laneidx = jnp.min(
            jnp.where(d == colmax, flat_ref[...], 3e8), axis=0, keepdims=True)
        mx = jnp.max(colmax, axis=1, keepdims=True)
        nxt = jnp.min(jnp.where(colmax == mx, laneidx, 3e8)).astype(jnp.int32)
        qx = sx_ref[nxt]
        qy = sy_ref[nxt]
        qz = sz_ref[nxt]
        idx_ref[i] = nxt
        qx_ref[i] = qx
        qy_ref[i] = qy
        qz_ref[i] = qz
        dx = px_ref[...] - qx
        dy = py_ref[...] - qy
        dz = pz_ref[...] - qz
        dn = dx * dx + dy * dy + dz * dz
        dists_ref[...] = jnp.minimum(d, dn)
        return carry

    jax.lax.fori_loop(0, m, body, 0)


def _fps(pos, n, m, rows):
    npad = rows * 128
    pt = jnp.zeros((3, npad), jnp.float32).at[:, :n].set(pos.T)
    px = pt[0].reshape(rows, 128)
    py = pt[1].reshape(rows, 128)
    pz = pt[2].reshape(rows, 128)
    idx, qx, qy, qz = pl.pallas_call(
        functools.partial(_fps_body, n, m),
        in_specs=[pl.BlockSpec(memory_space=pltpu.VMEM)] * 3
        + [pl.BlockSpec(memory_space=pltpu.SMEM)] * 3,
        out_shape=[
            jax.ShapeDtypeStruct((m,), jnp.int32),
            jax.ShapeDtypeStruct((m,), jnp.float32),
            jax.ShapeDtypeStruct((m,), jnp.float32),
            jax.ShapeDtypeStruct((m,), jnp.float32),
        ],
        out_specs=[pl.BlockSpec(memory_space=pltpu.SMEM)] * 4,
        scratch_shapes=[
            pltpu.VMEM((rows, 128), jnp.float32),
            pltpu.VMEM((rows, 128), jnp.float32),
        ],
    )(px, py, pz, pt[0], pt[1], pt[2])
    return idx, jnp.stack([qx, qy, qz], axis=1)


def _mlp_body(blkc, hp_ref, vf_ref, w1_ref, b1_ref, w2_ref, b2_ref,
              w3_ref, b3_ref, out_ref):
    h = hp_ref[...]
    h1 = jnp.maximum(
        jnp.dot(h, w1_ref[...], preferred_element_type=jnp.float32)
        + b1_ref[...], 0.0)
    h2 = jnp.maximum(
        jnp.dot(h1, w2_ref[...], preferred_element_type=jnp.float32)
        + b2_ref[...], 0.0)
    h3 = jnp.maximum(
        jnp.dot(h2, w3_ref[...], preferred_element_type=jnp.float32)
        + b3_ref[...], 0.0)
    h3r = h3.reshape(blkc, _K, h3.shape[-1])
    v = vf_ref[...]
    h3m = jnp.where(v[:, :, None] > 0, h3r, -1.0)
    r = jnp.max(h3m, axis=1)
    out_ref[...] = jnp.where(r < 0, 0.0, r)


def _mlp(hp, validf, W1p, b1, W2, b2, W3, b3, mp):
    blkc = 64
    grid = mp // blkc
    h3w = W3.shape[1]
    return pl.pallas_call(
        functools.partial(_mlp_body, blkc),
        grid=(grid,),
        in_specs=[
            pl.BlockSpec((blkc * _K, 256), lambda i: (i, 0)),
            pl.BlockSpec((blkc, _K), lambda i: (i, 0)),
            pl.BlockSpec((256, 128), lambda i: (0, 0)),
            pl.BlockSpec((1, 128), lambda i: (0, 0)),
            pl.BlockSpec((128, 128), lambda i: (0, 0)),
            pl.BlockSpec((1, 128), lambda i: (0, 0)),
            pl.BlockSpec((128, 256), lambda i: (0, 0)),
            pl.BlockSpec((1, 256), lambda i: (0, 0)),
        ],
        out_specs=pl.BlockSpec((blkc, h3w), lambda i: (i, 0)),
        out_shape=jax.ShapeDtypeStruct((mp, h3w), jnp.float32),
    )(hp, validf, W1p, b1, W2, b2, W3, b3)


def kernel(x, pos, batch, W1, b1, W2, b2, W3, b3):
    n, f = x.shape
    m = int(round(n * 0.25))
    rows = (n + 1023) // 1024 * 8
    idx, pos_q = _fps(pos, n, m, rows)

    d2 = jnp.sum((pos_q[:, None, :] - pos[None, :, :]) ** 2, axis=-1)
    d2m = jnp.where(d2 <= _R2, d2, jnp.inf)
    negv, nbr = jax.lax.top_k(-d2m, _K)
    valid = negv > -jnp.inf

    x_j = x[nbr].reshape(m * _K, f)
    rel = (pos[nbr] - pos_q[:, None, :]).reshape(m * _K, 3)

    mp = (m + 63) // 64 * 64
    hp = (jnp.zeros((mp * _K, 256), jnp.float32)
          .at[:m * _K, :f].set(x_j)
          .at[:m * _K, f:f + 3].set(rel))
    w1p = jnp.zeros((256, 128), jnp.float32).at[:f + 3, :].set(W1)
    validf = jnp.zeros((mp, _K), jnp.float32).at[:m].set(valid.astype(jnp.float32))

    out = _mlp(hp, validf, w1p, b1[None, :], W2, b2[None, :], W3, b3[None, :], mp)
    return (out[:m], pos_q, batch[idx])


# speedup vs baseline: 14.6603x; 4.9857x over previous
import functools

import jax
import jax.numpy as jnp
from jax import lax
from jax.experimental import pallas as pl
from jax.experimental.pallas import tpu as pltpu
from jax.experimental.pallas import tpu_sc as plsc

_R2 = 0.2 * 0.2
_K = 64


def _fps_body(n, m, px_ref, py_ref, pz_ref, sx_ref, sy_ref, sz_ref,
              idx_ref, qx_ref, qy_ref, qz_ref, dists_ref, flat_ref):
    shp = (px_ref.shape[0], px_ref.shape[1])
    flat = (jax.lax.broadcasted_iota(jnp.int32, shp, 0) * shp[1]
            + jax.lax.broadcasted_iota(jnp.int32, shp, 1))
    flat_ref[...] = flat.astype(jnp.float32)
    dists_ref[...] = jnp.where(flat < n, jnp.inf, -1.0)

    def body(i, carry):
        d = dists_ref[...]
        colmax = jnp.max(d, axis=0, keepdims=True)
        laneidx = jnp.min(
            jnp.where(d == colmax, flat_ref[...], 3e8), axis=0, keepdims=True)
        mx = jnp.max(colmax, axis=1, keepdims=True)
        nxt = jnp.min(jnp.where(colmax == mx, laneidx, 3e8)).astype(jnp.int32)
        qx = sx_ref[nxt]
        qy = sy_ref[nxt]
        qz = sz_ref[nxt]
        idx_ref[i] = nxt
        qx_ref[i] = qx
        qy_ref[i] = qy
        qz_ref[i] = qz
        dx = px_ref[...] - qx
        dy = py_ref[...] - qy
        dz = pz_ref[...] - qz
        dn = dx * dx + dy * dy + dz * dz
        dists_ref[...] = jnp.minimum(d, dn)
        return carry

    jax.lax.fori_loop(0, m, body, 0)


def _fps(pos, n, m, rows):
    npad = rows * 128
    pt = jnp.zeros((3, npad), jnp.float32).at[:, :n].set(pos.T)
    px = pt[0].reshape(rows, 128)
    py = pt[1].reshape(rows, 128)
    pz = pt[2].reshape(rows, 128)
    idx, qx, qy, qz = pl.pallas_call(
        functools.partial(_fps_body, n, m),
        in_specs=[pl.BlockSpec(memory_space=pltpu.VMEM)] * 3
        + [pl.BlockSpec(memory_space=pltpu.SMEM)] * 3,
        out_shape=[
            jax.ShapeDtypeStruct((m,), jnp.int32),
            jax.ShapeDtypeStruct((m,), jnp.float32),
            jax.ShapeDtypeStruct((m,), jnp.float32),
            jax.ShapeDtypeStruct((m,), jnp.float32),
        ],
        out_specs=[pl.BlockSpec(memory_space=pltpu.SMEM)] * 4,
        scratch_shapes=[
            pltpu.VMEM((rows, 128), jnp.float32),
            pltpu.VMEM((rows, 128), jnp.float32),
        ],
    )(px, py, pz, pt[0], pt[1], pt[2])
    return idx, jnp.stack([qx, qy, qz], axis=1)


def _d2_body(rows, qx_ref, qy_ref, qz_ref, px_ref, py_ref, pz_ref, out_ref):
    i = pl.program_id(0)
    px = px_ref[...]
    py = py_ref[...]
    pz = pz_ref[...]
    for r in range(8):
        qx = qx_ref[i * 8 + r]
        qy = qy_ref[i * 8 + r]
        qz = qz_ref[i * 8 + r]
        dx = px - qx
        dy = py - qy
        dz = pz - qz
        out_ref[r] = dx * dx + dy * dy + dz * dz


def _d2_matrix(px, py, pz, qxp, qyp, qzp, rows, mp):
    return pl.pallas_call(
        functools.partial(_d2_body, rows),
        grid=(mp // 8,),
        in_specs=[pl.BlockSpec(memory_space=pltpu.SMEM)] * 3
        + [pl.BlockSpec((rows, 128), lambda i: (0, 0))] * 3,
        out_specs=pl.BlockSpec((8, rows, 128), lambda i: (i, 0, 0)),
        out_shape=jax.ShapeDtypeStruct((mp, rows, 128), jnp.float32),
    )(qxp, qyp, qzp, px, py, pz)


def _sc_select(d2m, pxp, pyp, pzp, qxp, qyp, qzp, npad, mp):
    nw = 32
    rpw = mp // nw
    nv = npad // 16
    nbuck = 18
    r2 = jnp.float32(_R2)
    scale = jnp.float32(16.0 / _R2)
    mesh = plsc.VectorSubcoreMesh(core_axis_name="c", subcore_axis_name="s")

    @functools.partial(
        pl.kernel,
        mesh=mesh,
        compiler_params=pltpu.CompilerParams(needs_layout_passes=False),
        out_type=[
            jax.ShapeDtypeStruct((nw, rpw * _K), jnp.int32),
            jax.ShapeDtypeStruct((nw, rpw * _K), jnp.float32),
            jax.ShapeDtypeStruct((mp, _K, 8), jnp.float32),
        ],
        scratch_types=[
            pltpu.VMEM((npad + 128,), jnp.int32),
            pltpu.VMEM((npad + 128,), jnp.float32),
            pltpu.VMEM((npad,), jnp.float32),
            pltpu.VMEM((npad,), jnp.float32),
            pltpu.VMEM((npad,), jnp.float32),
            pltpu.VMEM((rpw,), jnp.float32),
            pltpu.VMEM((rpw,), jnp.float32),
            pltpu.VMEM((rpw,), jnp.float32),
            pltpu.VMEM((npad,), jnp.float32),
            pltpu.VMEM((npad,), jnp.int32),
            pltpu.VMEM((npad,), jnp.float32),
            pltpu.VMEM((npad,), jnp.int32),
            pltpu.VMEM((nbuck * 16,), jnp.int32),
            pltpu.VMEM((rpw * _K,), jnp.int32),
            pltpu.VMEM((rpw * _K,), jnp.float32),
            pltpu.VMEM((_K, 8), jnp.float32),
            pltpu.SemaphoreType.DMA,
        ],
    )
    def k(d2_hbm, px_hbm, py_hbm, pz_hbm, qx_hbm, qy_hbm, qz_hbm,
          nbr_hbm, val_hbm, rel_hbm,
          addrb, cd2e, pxv, pyv, pzv, qxv, qyv, qzv, cd2, cidx, bd2, bidx,
          hist, oidx, oval, relb, sem):
        wid = lax.axis_index("s") * 2 + (1 - lax.axis_index("c"))
        rbase = wid * rpw
        pltpu.sync_copy(px_hbm, pxv)
        pltpu.sync_copy(py_hbm, pyv)
        pltpu.sync_copy(pz_hbm, pzv)
        pltpu.sync_copy(qx_hbm.at[pl.ds(rbase, rpw)], qxv)
        pltpu.sync_copy(qy_hbm.at[pl.ds(rbase, rpw)], qyv)
        pltpu.sync_copy(qz_hbm.at[pl.ds(rbase, rpw)], qzv)

        iota = lax.iota(jnp.int32, 16)
        ones = jnp.ones((16,), jnp.int32)
        zerosf = jnp.zeros((16,), jnp.float32)
        inf16 = jnp.full((16,), jnp.inf, jnp.float32)
        big = jnp.full((16,), jnp.int32(2**30))

        r2sup = jnp.float32(_R2 * (1.0 + 1e-5))

        def row_fn(r, _):
            rsplat = jnp.full((16,), r, jnp.int32)
            qx = plsc.load_gather(qxv, [rsplat])
            qy = plsc.load_gather(qyv, [rsplat])
            qz = plsc.load_gather(qzv, [rsplat])
            obase = r * _K
            basea = (rbase + r) * npad

            for kk in range(nbuck):
                hist[pl.ds(kk * 16, 16)] = jnp.zeros((16,), jnp.int32)
            for kk in range(_K // 16):
                oidx[pl.ds(obase + kk * 16, 16)] = jnp.zeros((16,), jnp.int32)

            @plsc.parallel_loop(0, nv * 16, step=16, unroll=4,
                                carry=jnp.zeros((16,), jnp.int32))
            def scan1(base16, off):
                lanei = base16 + iota
                dx = pxv[pl.ds(base16, 16)] - qx
                dy = pyv[pl.ds(base16, 16)] - qy
                dz = pzv[pl.ds(base16, 16)] - qz
                d2s = dx * dx + dy * dy + dz * dz
                msk = d2s <= r2sup
                pc = plsc.cumsum(msk.astype(jnp.int32))
                dest = off + pc - 1
                plsc.store_scatter(addrb, [dest], basea + lanei, mask=msk)
                return off + plsc.all_reduce_population_count(msk)

            vs = jnp.max(scan1)

            basev = jnp.full((16,), basea, jnp.int32)
            for kk in range(8):
                addrb[pl.ds(vs + kk * 16, 16)] = basev
            ncg = (vs + 127) // 128

            def fire(c, _2):
                pltpu.async_copy(d2_hbm.at[addrb.at[pl.ds(c * 128, 128)]],
                                 cd2e.at[pl.ds(c * 128, 128)], sem)
                return 0
            lax.fori_loop(0, ncg, fire, 0)

            def drain(c, _2):
                pltpu.make_async_copy(
                    d2_hbm.at[addrb.at[pl.ds(c * 128, 128)]],
                    cd2e.at[pl.ds(c * 128, 128)], sem).wait()
                return 0
            lax.fori_loop(0, ncg, drain, 0)

            def scan2a(base16, off):
                lanei = base16 + iota
                lm = lanei < vs
                d2 = cd2e[pl.ds(base16, 16)]
                idxv = addrb[pl.ds(base16, 16)] - basea
                msk = (d2 <= r2) & lm
                pc = plsc.cumsum(msk.astype(jnp.int32))
                dest = off + pc - 1
                plsc.store_scatter(cd2, [dest], d2, mask=msk)
                plsc.store_scatter(cidx, [dest], idxv, mask=msk)
                b = jnp.minimum((d2 * scale).astype(jnp.int32), 16)
                plsc.addupdate_scatter(hist, [b * 16 + iota], ones, mask=msk)
                return off + plsc.all_reduce_population_count(msk)

            nsv = (vs + 15) // 16
            offv = plsc.parallel_loop(
                0, nsv * 16, step=16, unroll=2,
                carry=jnp.zeros((16,), jnp.int32))(scan2a)
            v = jnp.max(offv)
            nval = jnp.minimum(v, _K)

            for kk in range(_K // 16):
                lane = kk * 16 + iota
                oval[pl.ds(obase + kk * 16, 16)] = jnp.where(
                    lane < nval, 1.0, 0.0).astype(jnp.float32)

            @pl.when(v <= _K)
            def _small():
                def cpy(j, _2):
                    lane = j * 16 + iota
                    m = lane < v
                    idxv = cidx[pl.ds(j * 16, 16)]
                    plsc.store_scatter(oidx, [obase + lane], idxv, mask=m)
                    return 0
                lax.fori_loop(0, _K // 16, cpy, 0)

            @pl.when(v > _K)
            def _large():
                ncv = (v + 15) // 16
                cum = jnp.int32(0)
                bb = jnp.int32(-1)
                cbelow = jnp.int32(0)
                for kk in range(nbuck - 1):
                    t = jnp.sum(hist[pl.ds(kk * 16, 16)])
                    newcum = cum + t
                    hit = (bb < 0) & (newcum >= _K)
                    cbelow = jnp.where(hit, cum, cbelow)
                    bb = jnp.where(hit, kk, bb)
                    cum = newcum

                def scan2(base2, carry):
                    no, nb = carry
                    j = base2 // 16
                    lane = base2 + iota
                    lm = lane < v
                    d2 = cd2[pl.ds(j * 16, 16)]
                    idxv = cidx[pl.ds(j * 16, 16)]
                    b = jnp.minimum((d2 * scale).astype(jnp.int32), 16)
                    selm = (b < bb) & lm
                    pc = plsc.cumsum(selm.astype(jnp.int32))
                    plsc.store_scatter(oidx, [obase + no + pc - 1], idxv, mask=selm)
                    no = no + plsc.all_reduce_population_count(selm)
                    bm = (b == bb) & lm
                    pcb = plsc.cumsum(bm.astype(jnp.int32))
                    plsc.store_scatter(bd2, [nb + pcb - 1], d2, mask=bm)
                    plsc.store_scatter(bidx, [nb + pcb - 1], idxv, mask=bm)
                    nb = nb + plsc.all_reduce_population_count(bm)
                    return (no, nb)

                _, nbv = plsc.parallel_loop(
                    0, ncv * 16, step=16, unroll=2,
                    carry=(jnp.zeros((16,), jnp.int32),
                           jnp.zeros((16,), jnp.int32)))(scan2)
                nb = jnp.max(nbv)
                nbw = (nb + 15) // 16
                kprime = _K - cbelow

                def extract(t, _2):
                    def m1(j, mn):
                        lm = j * 16 + iota < nb
                        d2 = bd2[pl.ds(j * 16, 16)]
                        return jnp.minimum(mn, jnp.where(lm, d2, inf16))
                    mn = jnp.min(lax.fori_loop(0, nbw, m1, inf16))

                    def m2(j, mi):
                        lm = j * 16 + iota < nb
                        d2 = bd2[pl.ds(j * 16, 16)]
                        idxv = bidx[pl.ds(j * 16, 16)]
                        return jnp.minimum(
                            mi, jnp.where(lm & (d2 == mn), idxv, big))
                    mni = jnp.min(lax.fori_loop(0, nbw, m2, big))

                    def m3(j, _3):
                        lane = j * 16 + iota
                        lm = lane < nb
                        d2 = bd2[pl.ds(j * 16, 16)]
                        idxv = bidx[pl.ds(j * 16, 16)]
                        wm = lm & (d2 == mn) & (idxv == mni)
                        plsc.store_scatter(bd2, [lane], inf16, mask=wm)
                        return 0
                    lax.fori_loop(0, nbw, m3, 0)

                    plsc.store_scatter(
                        oidx, [jnp.full((16,), obase + cbelow + t, jnp.int32)],
                        jnp.full((16,), mni, jnp.int32), mask=iota == 0)
                    return 0

                lax.fori_loop(0, kprime, extract, 0)

            def relk(kk, _2):
                lane16 = kk * 16 + iota
                sel = plsc.load_gather(oidx, [obase + lane16])
                gx = plsc.load_gather(pxv, [sel]) - qx
                gy = plsc.load_gather(pyv, [sel]) - qy
                gz = plsc.load_gather(pzv, [sel]) - qz
                plsc.store_scatter(relb, [lane16, jnp.zeros((16,), jnp.int32)], gx)
                plsc.store_scatter(relb, [lane16, ones], gy)
                plsc.store_scatter(relb, [lane16, ones + ones], gz)
                for cc in range(3, 8):
                    plsc.store_scatter(
                        relb, [lane16, jnp.full((16,), cc, jnp.int32)], zerosf)
                return 0

            lax.fori_loop(0, _K // 16, relk, 0)
            pltpu.sync_copy(relb, rel_hbm.at[rbase + r])
            return 0

        lax.fori_loop(0, rpw, row_fn, 0)
        pltpu.sync_copy(oidx, nbr_hbm.at[wid])
        pltpu.sync_copy(oval, val_hbm.at[wid])

    return k(d2m, pxp, pyp, pzp, qxp, qyp, qzp)


def _sc_gather(x, idx_flat):
    n, d = x.shape
    b = idx_flat.shape[0]
    nw = 32
    bpw = b // nw
    nchunk = bpw // 128
    mesh = plsc.VectorSubcoreMesh(core_axis_name="c", subcore_axis_name="s")

    @functools.partial(
        pl.kernel,
        mesh=mesh,
        out_type=jax.ShapeDtypeStruct((b, d), jnp.float32),
        scratch_types=[
            pltpu.VMEM((nchunk, 128), jnp.int32),
            pltpu.VMEM((128, d), jnp.float32),
            pltpu.VMEM((128, d), jnp.float32),
            pltpu.SemaphoreType.DMA,
            pltpu.SemaphoreType.DMA,
        ],
    )
    def k(x_hbm, idx_hbm, out_hbm, idx_v, buf0, buf1, sem0, sem1):
        wid = lax.axis_index("s") * 2 + lax.axis_index("c")
        base = wid * bpw
        pltpu.sync_copy(idx_hbm.at[wid], idx_v)
        pltpu.async_copy(x_hbm.at[idx_v.at[0]], buf0, sem0)

        def body(j, _):
            c = j * 2
            pltpu.async_copy(x_hbm.at[idx_v.at[c + 1]], buf1, sem1)
            pltpu.make_async_copy(x_hbm.at[idx_v.at[c]], buf0, sem0).wait()
            pltpu.sync_copy(buf0, out_hbm.at[pl.ds(base + c * 128, 128)])

            @pl.when(c + 2 < nchunk)
            def _():
                pltpu.async_copy(x_hbm.at[idx_v.at[c + 2]], buf0, sem0)

            pltpu.make_async_copy(x_hbm.at[idx_v.at[c + 1]], buf1, sem1).wait()
            pltpu.sync_copy(buf1, out_hbm.at[pl.ds(base + (c + 1) * 128, 128)])
            return 0

        lax.fori_loop(0, nchunk // 2, body, 0)

    return k(x, idx_flat.reshape(nw, nchunk, 128))


def _xw_body(x_ref, w_ref, out_ref):
    out_ref[...] = jnp.dot(x_ref[...], w_ref[...],
                           preferred_element_type=jnp.float32)


def _xw(x, w1x):
    n, f = x.shape
    blk = 1000
    return pl.pallas_call(
        _xw_body,
        grid=(n // blk,),
        in_specs=[
            pl.BlockSpec((blk, f), lambda i: (i, 0)),
            pl.BlockSpec((f, 128), lambda i: (0, 0)),
        ],
        out_specs=pl.BlockSpec((blk, 128), lambda i: (i, 0)),
        out_shape=jax.ShapeDtypeStruct((n, 128), jnp.float32),
    )(x, w1x)


def _mlp_body(blkc, xj_ref, rel_ref, vf_ref, w1r_ref, b1_ref,
              w2_ref, b2_ref, w3_ref, b3_ref, out_ref):
    h1 = jnp.maximum(
        xj_ref[...]
        + jnp.dot(rel_ref[...], w1r_ref[...],
                  preferred_element_type=jnp.float32)
        + b1_ref[...], 0.0)
    h2 = jnp.maximum(
        jnp.dot(h1, w2_ref[...], preferred_element_type=jnp.float32)
        + b2_ref[...], 0.0)
    h3 = jnp.maximum(
        jnp.dot(h2, w3_ref[...], preferred_element_type=jnp.float32)
        + b3_ref[...], 0.0)
    h3r = h3.reshape(blkc, _K, h3.shape[-1])
    v = vf_ref[...]
    h3m = jnp.where(v[:, :, None] > 0, h3r, -1.0)
    r = jnp.max(h3m, axis=1)
    out_ref[...] = jnp.where(r < 0, 0.0, r)


def _mlp(xj, rel8, validf, W1r8, b1, W2, b2, W3, b3, mp):
    blkc = 64
    grid = mp // blkc
    h3w = W3.shape[1]
    f = xj.shape[1]
    return pl.pallas_call(
        functools.partial(_mlp_body, blkc),
        grid=(grid,),
        in_specs=[
            pl.BlockSpec((blkc * _K, f), lambda i: (i, 0)),
            pl.BlockSpec((blkc * _K, 8), lambda i: (i, 0)),
            pl.BlockSpec((blkc, _K), lambda i: (i, 0)),
            pl.BlockSpec((8, 128), lambda i: (0, 0)),
            pl.BlockSpec((1, 128), lambda i: (0, 0)),
            pl.BlockSpec((128, 128), lambda i: (0, 0)),
            pl.BlockSpec((1, 128), lambda i: (0, 0)),
            pl.BlockSpec((128, 256), lambda i: (0, 0)),
            pl.BlockSpec((1, 256), lambda i: (0, 0)),
        ],
        out_specs=pl.BlockSpec((blkc, h3w), lambda i: (i, 0)),
        out_shape=jax.ShapeDtypeStruct((mp, h3w), jnp.float32),
    )(xj, rel8, validf, W1r8, b1, W2, b2, W3, b3)


def kernel(x, pos, batch, W1, b1, W2, b2, W3, b3):
    n, f = x.shape
    m = int(round(n * 0.25))
    rows = (n + 1023) // 1024 * 8
    idx, pos_q = _fps(pos, n, m, rows)

    mp = 2560
    npad = rows * 128
    pad = jnp.float32(1e3)
    pxp = jnp.full((npad,), pad, jnp.float32).at[:n].set(pos[:, 0])
    pyp = jnp.full((npad,), pad, jnp.float32).at[:n].set(pos[:, 1])
    pzp = jnp.full((npad,), pad, jnp.float32).at[:n].set(pos[:, 2])
    qxp = jnp.full((mp,), pad, jnp.float32).at[:m].set(pos_q[:, 0])
    qyp = jnp.full((mp,), pad, jnp.float32).at[:m].set(pos_q[:, 1])
    qzp = jnp.full((mp,), pad, jnp.float32).at[:m].set(pos_q[:, 2])
    px = pxp.reshape(rows, 128)
    py = pyp.reshape(rows, 128)
    pz = pzp.reshape(rows, 128)
    d2m = _d2_matrix(px, py, pz, qxp, qyp, qzp, rows, mp).reshape(mp * npad)
    nbr32, val32, rel = _sc_select(d2m, pxp, pyp, pzp, qxp, qyp, qzp, npad, mp)

    u = _xw(x, W1[:f])
    uj = _sc_gather(u, nbr32.reshape(-1))
    rel8 = rel.reshape(mp * _K, 8)
    validf = val32.reshape(mp, _K)
    w1r8 = jnp.zeros((8, 128), jnp.float32).at[:3, :].set(W1[f:f + 3])

    out = _mlp(uj, rel8, validf, w1r8, b1[None, :], W2, b2[None, :],
               W3, b3[None, :], mp)
    return (out[:m], pos_q, batch[idx])


# speedup vs baseline: 25.1695x; 1.7168x over previous
import functools

import jax
import jax.numpy as jnp
from jax import lax
from jax.experimental import pallas as pl
from jax.experimental.pallas import tpu as pltpu
from jax.experimental.pallas import tpu_sc as plsc

_R2 = 0.2 * 0.2
_K = 64


def _fps_body(n, m, px_ref, py_ref, pz_ref, sx_ref, sy_ref, sz_ref,
              idx_ref, qx_ref, qy_ref, qz_ref, dists_ref, flat_ref):
    shp = (px_ref.shape[0], px_ref.shape[1])
    flat = (jax.lax.broadcasted_iota(jnp.int32, shp, 0) * shp[1]
            + jax.lax.broadcasted_iota(jnp.int32, shp, 1))
    flat_ref[...] = flat.astype(jnp.float32)
    dists_ref[...] = jnp.where(flat < n, jnp.inf, -1.0)

    def body(i, carry):
        d = dists_ref[...]
        colmax = jnp.max(d, axis=0, keepdims=True)
        laneidx = jnp.min(
            jnp.where(d == colmax, flat_ref[...], 3e8), axis=0, keepdims=True)
        mx = jnp.max(colmax, axis=1, keepdims=True)
        nxt = jnp.min(jnp.where(colmax == mx, laneidx, 3e8)).astype(jnp.int32)
        qx = sx_ref[nxt]
        qy = sy_ref[nxt]
        qz = sz_ref[nxt]
        idx_ref[i] = nxt
        qx_ref[i] = qx
        qy_ref[i] = qy
        qz_ref[i] = qz
        dx = px_ref[...] - qx
        dy = py_ref[...] - qy
        dz = pz_ref[...] - qz
        dn = dx * dx + dy * dy + dz * dz
        dists_ref[...] = jnp.minimum(d, dn)
        return carry

    jax.lax.fori_loop(0, m, body, 0)


def _fps(pos, n, m, rows):
    npad = rows * 128
    pt = jnp.zeros((3, npad), jnp.float32).at[:, :n].set(pos.T)
    px = pt[0].reshape(rows, 128)
    py = pt[1].reshape(rows, 128)
    pz = pt[2].reshape(rows, 128)
    idx, qx, qy, qz = pl.pallas_call(
        functools.partial(_fps_body, n, m),
        in_specs=[pl.BlockSpec(memory_space=pltpu.VMEM)] * 3
        + [pl.BlockSpec(memory_space=pltpu.SMEM)] * 3,
        out_shape=[
            jax.ShapeDtypeStruct((m,), jnp.int32),
            jax.ShapeDtypeStruct((m,), jnp.float32),
            jax.ShapeDtypeStruct((m,), jnp.float32),
            jax.ShapeDtypeStruct((m,), jnp.float32),
        ],
        out_specs=[pl.BlockSpec(memory_space=pltpu.SMEM)] * 4,
        scratch_shapes=[
            pltpu.VMEM((rows, 128), jnp.float32),
            pltpu.VMEM((rows, 128), jnp.float32),
        ],
    )(px, py, pz, pt[0], pt[1], pt[2])
    return idx, jnp.stack([qx, qy, qz], axis=1)


def _d2_body(rows, qx_ref, qy_ref, qz_ref, px_ref, py_ref, pz_ref, out_ref):
    i = pl.program_id(0)
    px = px_ref[...]
    py = py_ref[...]
    pz = pz_ref[...]
    for r in range(8):
        qx = qx_ref[i * 8 + r]
        qy = qy_ref[i * 8 + r]
        qz = qz_ref[i * 8 + r]
        dx = px - qx
        dy = py - qy
        dz = pz - qz
        out_ref[r] = dx * dx + dy * dy + dz * dz


def _d2_matrix(px, py, pz, qxp, qyp, qzp, rows, mp):
    return pl.pallas_call(
        functools.partial(_d2_body, rows),
        grid=(mp // 8,),
        in_specs=[pl.BlockSpec(memory_space=pltpu.SMEM)] * 3
        + [pl.BlockSpec((rows, 128), lambda i: (0, 0))] * 3,
        out_specs=pl.BlockSpec((8, rows, 128), lambda i: (i, 0, 0)),
        out_shape=jax.ShapeDtypeStruct((mp, rows, 128), jnp.float32),
    )(qxp, qyp, qzp, px, py, pz)


def _sc_select(d2m, pxp, pyp, pzp, qxp, qyp, qzp, npad, mp):
    nw = 32
    rpw = mp // nw
    nv = npad // 16
    nbuck = 18
    r2 = jnp.float32(_R2)
    scale = jnp.float32(16.0 / _R2)
    mesh = plsc.VectorSubcoreMesh(core_axis_name="c", subcore_axis_name="s")

    @functools.partial(
        pl.kernel,
        mesh=mesh,
        compiler_params=pltpu.CompilerParams(needs_layout_passes=False),
        out_type=[
            jax.ShapeDtypeStruct((nw, rpw * _K), jnp.int32),
            jax.ShapeDtypeStruct((nw, rpw * _K), jnp.float32),
            jax.ShapeDtypeStruct((mp, _K, 8), jnp.float32),
        ],
        scratch_types=[
            pltpu.VMEM((npad + 128,), jnp.int32),
            pltpu.VMEM((npad + 128,), jnp.float32),
            pltpu.VMEM((npad,), jnp.float32),
            pltpu.VMEM((npad,), jnp.float32),
            pltpu.VMEM((npad,), jnp.float32),
            pltpu.VMEM((mp,), jnp.float32),
            pltpu.VMEM((mp,), jnp.float32),
            pltpu.VMEM((mp,), jnp.float32),
            pltpu.VMEM((npad,), jnp.float32),
            pltpu.VMEM((npad,), jnp.int32),
            pltpu.VMEM((npad,), jnp.float32),
            pltpu.VMEM((npad,), jnp.int32),
            pltpu.VMEM((nbuck * 16,), jnp.int32),
            pltpu.VMEM((rpw * _K,), jnp.int32),
            pltpu.VMEM((rpw * _K,), jnp.float32),
            pltpu.VMEM((_K, 8), jnp.float32),
            pltpu.SemaphoreType.DMA,
        ],
    )
    def k(d2_hbm, px_hbm, py_hbm, pz_hbm, qx_hbm, qy_hbm, qz_hbm,
          nbr_hbm, val_hbm, rel_hbm,
          addrb, cd2e, pxv, pyv, pzv, qxv, qyv, qzv, cd2, cidx, bd2, bidx,
          hist, oidx, oval, relb, sem):
        wid = lax.axis_index("s") * 2 + (1 - lax.axis_index("c"))
        rbase = wid * rpw
        pltpu.sync_copy(px_hbm, pxv)
        pltpu.sync_copy(py_hbm, pyv)
        pltpu.sync_copy(pz_hbm, pzv)
        pltpu.sync_copy(qx_hbm, qxv)
        pltpu.sync_copy(qy_hbm, qyv)
        pltpu.sync_copy(qz_hbm, qzv)

        iota = lax.iota(jnp.int32, 16)
        ones = jnp.ones((16,), jnp.int32)
        zerosf = jnp.zeros((16,), jnp.float32)
        inf16 = jnp.full((16,), jnp.inf, jnp.float32)
        big = jnp.full((16,), jnp.int32(2**30))

        r2sup = jnp.float32(_R2 * (1.0 + 1e-5))

        def row_fn(r, _):
            grow = r * nw + wid
            rsplat = jnp.full((16,), grow, jnp.int32)
            qx = plsc.load_gather(qxv, [rsplat])
            qy = plsc.load_gather(qyv, [rsplat])
            qz = plsc.load_gather(qzv, [rsplat])
            obase = r * _K
            basea = grow * npad

            for kk in range(nbuck):
                hist[pl.ds(kk * 16, 16)] = jnp.zeros((16,), jnp.int32)
            for kk in range(_K // 16):
                oidx[pl.ds(obase + kk * 16, 16)] = jnp.zeros((16,), jnp.int32)

            @plsc.parallel_loop(0, nv * 16, step=16, unroll=4,
                                carry=jnp.zeros((16,), jnp.int32))
            def scan1(base16, off):
                lanei = base16 + iota
                dx = pxv[pl.ds(base16, 16)] - qx
                dy = pyv[pl.ds(base16, 16)] - qy
                dz = pzv[pl.ds(base16, 16)] - qz
                d2s = dx * dx + dy * dy + dz * dz
                msk = d2s <= r2sup
                pc = plsc.cumsum(msk.astype(jnp.int32))
                dest = off + pc - 1
                plsc.store_scatter(addrb, [dest], basea + lanei, mask=msk)
                return off + plsc.all_reduce_population_count(msk)

            vs = jnp.max(scan1)

            basev = jnp.full((16,), basea, jnp.int32)
            for kk in range(8):
                addrb[pl.ds(vs + kk * 16, 16)] = basev
            ncg = (vs + 127) // 128

            def fire(c, _2):
                pltpu.async_copy(d2_hbm.at[addrb.at[pl.ds(c * 128, 128)]],
                                 cd2e.at[pl.ds(c * 128, 128)], sem)
                return 0
            lax.fori_loop(0, ncg, fire, 0)

            def drain(c, _2):
                pltpu.make_async_copy(
                    d2_hbm.at[addrb.at[pl.ds(c * 128, 128)]],
                    cd2e.at[pl.ds(c * 128, 128)], sem).wait()
                return 0
            lax.fori_loop(0, ncg, drain, 0)

            def scan2a(base16, off):
                lanei = base16 + iota
                lm = lanei < vs
                d2 = cd2e[pl.ds(base16, 16)]
                idxv = addrb[pl.ds(base16, 16)] - basea
                msk = (d2 <= r2) & lm
                pc = plsc.cumsum(msk.astype(jnp.int32))
                dest = off + pc - 1
                plsc.store_scatter(cd2, [dest], d2, mask=msk)
                plsc.store_scatter(cidx, [dest], idxv, mask=msk)
                b = jnp.minimum((d2 * scale).astype(jnp.int32), 16)
                plsc.addupdate_scatter(hist, [b * 16 + iota], ones, mask=msk)
                return off + plsc.all_reduce_population_count(msk)

            nsv = (vs + 15) // 16
            offv = plsc.parallel_loop(
                0, nsv * 16, step=16, unroll=2,
                carry=jnp.zeros((16,), jnp.int32))(scan2a)
            v = jnp.max(offv)
            nval = jnp.minimum(v, _K)

            for kk in range(_K // 16):
                lane = kk * 16 + iota
                oval[pl.ds(obase + kk * 16, 16)] = jnp.where(
                    lane < nval, 1.0, 0.0).astype(jnp.float32)

            @pl.when(v <= _K)
            def _small():
                def cpy(j, _2):
                    lane = j * 16 + iota
                    m = lane < v
                    idxv = cidx[pl.ds(j * 16, 16)]
                    plsc.store_scatter(oidx, [obase + lane], idxv, mask=m)
                    return 0
                lax.fori_loop(0, _K // 16, cpy, 0)

            @pl.when(v > _K)
            def _large():
                ncv = (v + 15) // 16
                cum = jnp.int32(0)
                bb = jnp.int32(-1)
                cbelow = jnp.int32(0)
                for kk in range(nbuck - 1):
                    t = jnp.sum(hist[pl.ds(kk * 16, 16)])
                    newcum = cum + t
                    hit = (bb < 0) & (newcum >= _K)
                    cbelow = jnp.where(hit, cum, cbelow)
                    bb = jnp.where(hit, kk, bb)
                    cum = newcum

                def scan2(base2, carry):
                    no, nb = carry
                    j = base2 // 16
                    lane = base2 + iota
                    lm = lane < v
                    d2 = cd2[pl.ds(j * 16, 16)]
                    idxv = cidx[pl.ds(j * 16, 16)]
                    b = jnp.minimum((d2 * scale).astype(jnp.int32), 16)
                    selm = (b < bb) & lm
                    pc = plsc.cumsum(selm.astype(jnp.int32))
                    plsc.store_scatter(oidx, [obase + no + pc - 1], idxv, mask=selm)
                    no = no + plsc.all_reduce_population_count(selm)
                    bm = (b == bb) & lm
                    pcb = plsc.cumsum(bm.astype(jnp.int32))
                    plsc.store_scatter(bd2, [nb + pcb - 1], d2, mask=bm)
                    plsc.store_scatter(bidx, [nb + pcb - 1], idxv, mask=bm)
                    nb = nb + plsc.all_reduce_population_count(bm)
                    return (no, nb)

                _, nbv = plsc.parallel_loop(
                    0, ncv * 16, step=16, unroll=2,
                    carry=(jnp.zeros((16,), jnp.int32),
                           jnp.zeros((16,), jnp.int32)))(scan2)
                nb = jnp.max(nbv)
                nbw = (nb + 15) // 16
                kprime = _K - cbelow

                def extract(t, _2):
                    def m1(j, mn):
                        lm = j * 16 + iota < nb
                        d2 = bd2[pl.ds(j * 16, 16)]
                        return jnp.minimum(mn, jnp.where(lm, d2, inf16))
                    mn = jnp.min(lax.fori_loop(0, nbw, m1, inf16))

                    def m2(j, mi):
                        lm = j * 16 + iota < nb
                        d2 = bd2[pl.ds(j * 16, 16)]
                        idxv = bidx[pl.ds(j * 16, 16)]
                        return jnp.minimum(
                            mi, jnp.where(lm & (d2 == mn), idxv, big))
                    mni = jnp.min(lax.fori_loop(0, nbw, m2, big))

                    def m3(j, _3):
                        lane = j * 16 + iota
                        lm = lane < nb
                        d2 = bd2[pl.ds(j * 16, 16)]
                        idxv = bidx[pl.ds(j * 16, 16)]
                        wm = lm & (d2 == mn) & (idxv == mni)
                        plsc.store_scatter(bd2, [lane], inf16, mask=wm)
                        return 0
                    lax.fori_loop(0, nbw, m3, 0)

                    plsc.store_scatter(
                        oidx, [jnp.full((16,), obase + cbelow + t, jnp.int32)],
                        jnp.full((16,), mni, jnp.int32), mask=iota == 0)
                    return 0

                lax.fori_loop(0, kprime, extract, 0)

            def relk(kk, _2):
                lane16 = kk * 16 + iota
                sel = plsc.load_gather(oidx, [obase + lane16])
                gx = plsc.load_gather(pxv, [sel]) - qx
                gy = plsc.load_gather(pyv, [sel]) - qy
                gz = plsc.load_gather(pzv, [sel]) - qz
                plsc.store_scatter(relb, [lane16, jnp.zeros((16,), jnp.int32)], gx)
                plsc.store_scatter(relb, [lane16, ones], gy)
                plsc.store_scatter(relb, [lane16, ones + ones], gz)
                for cc in range(3, 8):
                    plsc.store_scatter(
                        relb, [lane16, jnp.full((16,), cc, jnp.int32)], zerosf)
                return 0

            lax.fori_loop(0, _K // 16, relk, 0)
            pltpu.sync_copy(relb, rel_hbm.at[grow])
            return 0

        lax.fori_loop(0, rpw, row_fn, 0)
        pltpu.sync_copy(oidx, nbr_hbm.at[wid])
        pltpu.sync_copy(oval, val_hbm.at[wid])

    return k(d2m, pxp, pyp, pzp, qxp, qyp, qzp)


def _sc_gather(x, idx_flat):
    n, d = x.shape
    b = idx_flat.shape[0]
    nw = 32
    bpw = b // nw
    nchunk = bpw // 128
    mesh = plsc.VectorSubcoreMesh(core_axis_name="c", subcore_axis_name="s")

    @functools.partial(
        pl.kernel,
        mesh=mesh,
        out_type=jax.ShapeDtypeStruct((b, d), jnp.float32),
        scratch_types=[
            pltpu.VMEM((nchunk, 128), jnp.int32),
            pltpu.VMEM((128, d), jnp.float32),
            pltpu.VMEM((128, d), jnp.float32),
            pltpu.SemaphoreType.DMA,
            pltpu.SemaphoreType.DMA,
        ],
    )
    def k(x_hbm, idx_hbm, out_hbm, idx_v, buf0, buf1, sem0, sem1):
        wid = lax.axis_index("s") * 2 + lax.axis_index("c")
        base = wid * bpw
        pltpu.sync_copy(idx_hbm.at[wid], idx_v)
        pltpu.async_copy(x_hbm.at[idx_v.at[0]], buf0, sem0)

        def body(j, _):
            c = j * 2
            pltpu.async_copy(x_hbm.at[idx_v.at[c + 1]], buf1, sem1)
            pltpu.make_async_copy(x_hbm.at[idx_v.at[c]], buf0, sem0).wait()
            pltpu.sync_copy(buf0, out_hbm.at[pl.ds(base + c * 128, 128)])

            @pl.when(c + 2 < nchunk)
            def _():
                pltpu.async_copy(x_hbm.at[idx_v.at[c + 2]], buf0, sem0)

            pltpu.make_async_copy(x_hbm.at[idx_v.at[c + 1]], buf1, sem1).wait()
            pltpu.sync_copy(buf1, out_hbm.at[pl.ds(base + (c + 1) * 128, 128)])
            return 0

        lax.fori_loop(0, nchunk // 2, body, 0)

    return k(x, idx_flat.reshape(nw, nchunk, 128))


def _xw_body(x_ref, w_ref, out_ref):
    out_ref[...] = jnp.dot(x_ref[...], w_ref[...],
                           preferred_element_type=jnp.float32)


def _xw(x, w1x):
    n, f = x.shape
    blk = 1000
    return pl.pallas_call(
        _xw_body,
        grid=(n // blk,),
        in_specs=[
            pl.BlockSpec((blk, f), lambda i: (i, 0)),
            pl.BlockSpec((f, 128), lambda i: (0, 0)),
        ],
        out_specs=pl.BlockSpec((blk, 128), lambda i: (i, 0)),
        out_shape=jax.ShapeDtypeStruct((n, 128), jnp.float32),
    )(x, w1x)


def _mlp_body(blkc, xj_ref, rel_ref, vf_ref, w1r_ref, b1_ref,
              w2_ref, b2_ref, w3_ref, b3_ref, out_ref):
    h1 = jnp.maximum(
        xj_ref[...]
        + jnp.dot(rel_ref[...], w1r_ref[...],
                  preferred_element_type=jnp.float32)
        + b1_ref[...], 0.0)
    h2 = jnp.maximum(
        jnp.dot(h1, w2_ref[...], preferred_element_type=jnp.float32)
        + b2_ref[...], 0.0)
    h3 = jnp.maximum(
        jnp.dot(h2, w3_ref[...], preferred_element_type=jnp.float32)
        + b3_ref[...], 0.0)
    h3r = h3.reshape(blkc, _K, h3.shape[-1])
    v = vf_ref[...]
    h3m = jnp.where(v[:, :, None] > 0, h3r, -1.0)
    r = jnp.max(h3m, axis=1)
    out_ref[...] = jnp.where(r < 0, 0.0, r)


def _mlp(xj, rel8, validf, W1r8, b1, W2, b2, W3, b3, mp):
    blkc = 64
    grid = mp // blkc
    h3w = W3.shape[1]
    f = xj.shape[1]
    return pl.pallas_call(
        functools.partial(_mlp_body, blkc),
        grid=(grid,),
        in_specs=[
            pl.BlockSpec((blkc * _K, f), lambda i: (i, 0)),
            pl.BlockSpec((blkc * _K, 8), lambda i: (i, 0)),
            pl.BlockSpec((blkc, _K), lambda i: (i, 0)),
            pl.BlockSpec((8, 128), lambda i: (0, 0)),
            pl.BlockSpec((1, 128), lambda i: (0, 0)),
            pl.BlockSpec((128, 128), lambda i: (0, 0)),
            pl.BlockSpec((1, 128), lambda i: (0, 0)),
            pl.BlockSpec((128, 256), lambda i: (0, 0)),
            pl.BlockSpec((1, 256), lambda i: (0, 0)),
        ],
        out_specs=pl.BlockSpec((blkc, h3w), lambda i: (i, 0)),
        out_shape=jax.ShapeDtypeStruct((mp, h3w), jnp.float32),
    )(xj, rel8, validf, W1r8, b1, W2, b2, W3, b3)


def kernel(x, pos, batch, W1, b1, W2, b2, W3, b3):
    n, f = x.shape
    m = int(round(n * 0.25))
    rows = (n + 1023) // 1024 * 8
    idx, pos_q = _fps(pos, n, m, rows)

    mp = 2560
    npad = rows * 128
    pad = jnp.float32(1e3)
    pxp = jnp.full((npad,), pad, jnp.float32).at[:n].set(pos[:, 0])
    pyp = jnp.full((npad,), pad, jnp.float32).at[:n].set(pos[:, 1])
    pzp = jnp.full((npad,), pad, jnp.float32).at[:n].set(pos[:, 2])
    qxp = jnp.full((mp,), pad, jnp.float32).at[:m].set(pos_q[:, 0])
    qyp = jnp.full((mp,), pad, jnp.float32).at[:m].set(pos_q[:, 1])
    qzp = jnp.full((mp,), pad, jnp.float32).at[:m].set(pos_q[:, 2])
    px = pxp.reshape(rows, 128)
    py = pyp.reshape(rows, 128)
    pz = pzp.reshape(rows, 128)
    d2m = _d2_matrix(px, py, pz, qxp, qyp, qzp, rows, mp).reshape(mp * npad)
    nbr32, val32, rel = _sc_select(d2m, pxp, pyp, pzp, qxp, qyp, qzp, npad, mp)

    u = _xw(x, W1[:f])
    nbr_g = nbr32.reshape(32, mp // 32, _K).transpose(1, 0, 2)
    uj = _sc_gather(u, nbr_g.reshape(-1))
    rel8 = rel.reshape(mp * _K, 8)
    validf = val32.reshape(32, mp // 32, _K).transpose(1, 0, 2).reshape(mp, _K)
    w1r8 = jnp.zeros((8, 128), jnp.float32).at[:3, :].set(W1[f:f + 3])

    out = _mlp(uj, rel8, validf, w1r8, b1[None, :], W2, b2[None, :],
               W3, b3[None, :], mp)
    return (out[:m], pos_q, batch[idx])


# speedup vs baseline: 27.9746x; 1.1114x over previous
import functools

import jax
import jax.numpy as jnp
from jax import lax
from jax.experimental import pallas as pl
from jax.experimental.pallas import tpu as pltpu
from jax.experimental.pallas import tpu_sc as plsc

_R2 = 0.2 * 0.2
_K = 64


def _fps_body(n, m, px_ref, py_ref, pz_ref, sx_ref, sy_ref, sz_ref,
              idx_ref, qx_ref, qy_ref, qz_ref, dists_ref, flat_ref):
    shp = (px_ref.shape[0], px_ref.shape[1])
    flat = (jax.lax.broadcasted_iota(jnp.int32, shp, 0) * shp[1]
            + jax.lax.broadcasted_iota(jnp.int32, shp, 1))
    flat_ref[...] = flat.astype(jnp.float32)
    dists_ref[...] = jnp.where(flat < n, jnp.inf, -1.0)

    def body(i, carry):
        d = dists_ref[...]
        colmax = jnp.max(d, axis=0, keepdims=True)
        laneidx = jnp.min(
            jnp.where(d == colmax, flat_ref[...], 3e8), axis=0, keepdims=True)
        mx = jnp.max(colmax, axis=1, keepdims=True)
        nxt = jnp.min(jnp.where(colmax == mx, laneidx, 3e8)).astype(jnp.int32)
        qx = sx_ref[nxt]
        qy = sy_ref[nxt]
        qz = sz_ref[nxt]
        idx_ref[i] = nxt
        qx_ref[i] = qx
        qy_ref[i] = qy
        qz_ref[i] = qz
        dx = px_ref[...] - qx
        dy = py_ref[...] - qy
        dz = pz_ref[...] - qz
        dn = dx * dx + dy * dy + dz * dz
        dists_ref[...] = jnp.minimum(d, dn)
        return carry

    jax.lax.fori_loop(0, m, body, 0)


def _fps(pos, n, m, rows):
    npad = rows * 128
    pt = jnp.zeros((3, npad), jnp.float32).at[:, :n].set(pos.T)
    px = pt[0].reshape(rows, 128)
    py = pt[1].reshape(rows, 128)
    pz = pt[2].reshape(rows, 128)
    idx, qx, qy, qz = pl.pallas_call(
        functools.partial(_fps_body, n, m),
        in_specs=[pl.BlockSpec(memory_space=pltpu.VMEM)] * 3
        + [pl.BlockSpec(memory_space=pltpu.SMEM)] * 3,
        out_shape=[
            jax.ShapeDtypeStruct((m,), jnp.int32),
            jax.ShapeDtypeStruct((m,), jnp.float32),
            jax.ShapeDtypeStruct((m,), jnp.float32),
            jax.ShapeDtypeStruct((m,), jnp.float32),
        ],
        out_specs=[pl.BlockSpec(memory_space=pltpu.SMEM)] * 4,
        scratch_shapes=[
            pltpu.VMEM((rows, 128), jnp.float32),
            pltpu.VMEM((rows, 128), jnp.float32),
        ],
    )(px, py, pz, pt[0], pt[1], pt[2])
    return idx, jnp.stack([qx, qy, qz], axis=1)


def _d2_body(rows, qx_ref, qy_ref, qz_ref, px_ref, py_ref, pz_ref, out_ref):
    i = pl.program_id(0)
    px = px_ref[...]
    py = py_ref[...]
    pz = pz_ref[...]
    for r in range(8):
        qx = qx_ref[i * 8 + r]
        qy = qy_ref[i * 8 + r]
        qz = qz_ref[i * 8 + r]
        dx = px - qx
        dy = py - qy
        dz = pz - qz
        out_ref[r] = dx * dx + dy * dy + dz * dz


def _d2_matrix(px, py, pz, qxp, qyp, qzp, rows, mp):
    return pl.pallas_call(
        functools.partial(_d2_body, rows),
        grid=(mp // 8,),
        in_specs=[pl.BlockSpec(memory_space=pltpu.SMEM)] * 3
        + [pl.BlockSpec((rows, 128), lambda i: (0, 0))] * 3,
        out_specs=pl.BlockSpec((8, rows, 128), lambda i: (i, 0, 0)),
        out_shape=jax.ShapeDtypeStruct((mp, rows, 128), jnp.float32),
    )(qxp, qyp, qzp, px, py, pz)


def _sc_select(pxp, pyp, pzp, qxp, qyp, qzp, npad, mp):
    nw = 32
    rpw = mp // nw
    nv = npad // 16
    nbuck = 18
    r2 = jnp.float32(_R2)
    scale = jnp.float32(16.0 / _R2)
    mesh = plsc.VectorSubcoreMesh(core_axis_name="c", subcore_axis_name="s")

    @functools.partial(
        pl.kernel,
        mesh=mesh,
        compiler_params=pltpu.CompilerParams(needs_layout_passes=False),
        out_type=[
            jax.ShapeDtypeStruct((nw, rpw * _K), jnp.int32),
            jax.ShapeDtypeStruct((nw, rpw * _K), jnp.float32),
            jax.ShapeDtypeStruct((mp, _K, 8), jnp.float32),
        ],
        scratch_types=[
            pltpu.VMEM((npad,), jnp.float32),
            pltpu.VMEM((npad,), jnp.float32),
            pltpu.VMEM((npad,), jnp.float32),
            pltpu.VMEM((npad,), jnp.float32),
            pltpu.VMEM((npad,), jnp.float32),
            pltpu.VMEM((npad,), jnp.float32),
            pltpu.VMEM((mp,), jnp.float32),
            pltpu.VMEM((mp,), jnp.float32),
            pltpu.VMEM((mp,), jnp.float32),
            pltpu.VMEM((npad,), jnp.float32),
            pltpu.VMEM((npad,), jnp.int32),
            pltpu.VMEM((npad,), jnp.float32),
            pltpu.VMEM((npad,), jnp.int32),
            pltpu.VMEM((nbuck * 16,), jnp.int32),
            pltpu.VMEM((rpw * _K,), jnp.int32),
            pltpu.VMEM((rpw * _K,), jnp.float32),
            pltpu.VMEM((_K, 8), jnp.float32),
            pltpu.SemaphoreType.DMA,
        ],
    )
    def k(px_hbm, py_hbm, pz_hbm, qx_hbm, qy_hbm, qz_hbm,
          nbr_hbm, val_hbm, rel_hbm,
          sqx, sqy, sqz, pxv, pyv, pzv, qxv, qyv, qzv, cd2, cidx, bd2, bidx,
          hist, oidx, oval, relb, sem):
        wid = lax.axis_index("s") * 2 + (1 - lax.axis_index("c"))
        rbase = wid * rpw
        pltpu.sync_copy(px_hbm, pxv)
        pltpu.sync_copy(py_hbm, pyv)
        pltpu.sync_copy(pz_hbm, pzv)
        pltpu.sync_copy(qx_hbm, qxv)
        pltpu.sync_copy(qy_hbm, qyv)
        pltpu.sync_copy(qz_hbm, qzv)

        iota = lax.iota(jnp.int32, 16)
        ones = jnp.ones((16,), jnp.int32)
        zerosf = jnp.zeros((16,), jnp.float32)
        inf16 = jnp.full((16,), jnp.inf, jnp.float32)
        big = jnp.full((16,), jnp.int32(2**30))

        r2sup = jnp.float32(_R2 * (1.0 + 1e-5))

        def row_fn(r, _):
            grow = r * nw + wid
            rsplat = jnp.full((16,), grow, jnp.int32)
            qx = plsc.load_gather(qxv, [rsplat])
            qy = plsc.load_gather(qyv, [rsplat])
            qz = plsc.load_gather(qzv, [rsplat])
            obase = r * _K
            basea = grow * npad

            for kk in range(nbuck):
                hist[pl.ds(kk * 16, 16)] = jnp.zeros((16,), jnp.int32)
            for kk in range(_K // 16):
                oidx[pl.ds(obase + kk * 16, 16)] = jnp.zeros((16,), jnp.int32)

            @plsc.parallel_loop(0, nv * 16, step=16, unroll=4)
            def passa(base16):
                dx = pxv[pl.ds(base16, 16)] - qx
                dy = pyv[pl.ds(base16, 16)] - qy
                dz = pzv[pl.ds(base16, 16)] - qz
                sqx[pl.ds(base16, 16)] = dx * dx
                sqy[pl.ds(base16, 16)] = dy * dy
                sqz[pl.ds(base16, 16)] = dz * dz

            @plsc.parallel_loop(0, nv * 16, step=16, unroll=4,
                                carry=jnp.zeros((16,), jnp.int32))
            def scan1(base16, off):
                lanei = base16 + iota
                d2 = (sqx[pl.ds(base16, 16)] + sqy[pl.ds(base16, 16)]
                      ) + sqz[pl.ds(base16, 16)]
                msk = d2 <= r2
                pc = plsc.cumsum(msk.astype(jnp.int32))
                dest = off + pc - 1
                plsc.store_scatter(cd2, [dest], d2, mask=msk)
                plsc.store_scatter(cidx, [dest], lanei, mask=msk)
                b = jnp.minimum((d2 * scale).astype(jnp.int32), 16)
                plsc.addupdate_scatter(hist, [b * 16 + iota], ones, mask=msk)
                return off + plsc.all_reduce_population_count(msk)

            offv = scan1
            v = jnp.max(offv)
            nval = jnp.minimum(v, _K)

            for kk in range(_K // 16):
                lane = kk * 16 + iota
                oval[pl.ds(obase + kk * 16, 16)] = jnp.where(
                    lane < nval, 1.0, 0.0).astype(jnp.float32)

            @pl.when(v <= _K)
            def _small():
                def cpy(j, _2):
                    lane = j * 16 + iota
                    m = lane < v
                    idxv = cidx[pl.ds(j * 16, 16)]
                    plsc.store_scatter(oidx, [obase + lane], idxv, mask=m)
                    return 0
                lax.fori_loop(0, _K // 16, cpy, 0)

            @pl.when(v > _K)
            def _large():
                ncv = (v + 15) // 16
                cum = jnp.int32(0)
                bb = jnp.int32(-1)
                cbelow = jnp.int32(0)
                for kk in range(nbuck - 1):
                    t = jnp.sum(hist[pl.ds(kk * 16, 16)])
                    newcum = cum + t
                    hit = (bb < 0) & (newcum >= _K)
                    cbelow = jnp.where(hit, cum, cbelow)
                    bb = jnp.where(hit, kk, bb)
                    cum = newcum

                def scan2(base2, carry):
                    no, nb = carry
                    j = base2 // 16
                    lane = base2 + iota
                    lm = lane < v
                    d2 = cd2[pl.ds(j * 16, 16)]
                    idxv = cidx[pl.ds(j * 16, 16)]
                    b = jnp.minimum((d2 * scale).astype(jnp.int32), 16)
                    selm = (b < bb) & lm
                    pc = plsc.cumsum(selm.astype(jnp.int32))
                    plsc.store_scatter(oidx, [obase + no + pc - 1], idxv, mask=selm)
                    no = no + plsc.all_reduce_population_count(selm)
                    bm = (b == bb) & lm
                    pcb = plsc.cumsum(bm.astype(jnp.int32))
                    plsc.store_scatter(bd2, [nb + pcb - 1], d2, mask=bm)
                    plsc.store_scatter(bidx, [nb + pcb - 1], idxv, mask=bm)
                    nb = nb + plsc.all_reduce_population_count(bm)
                    return (no, nb)

                _, nbv = plsc.parallel_loop(
                    0, ncv * 16, step=16, unroll=2,
                    carry=(jnp.zeros((16,), jnp.int32),
                           jnp.zeros((16,), jnp.int32)))(scan2)
                nb = jnp.max(nbv)
                nbw = (nb + 15) // 16
                kprime = _K - cbelow

                def extract(t, _2):
                    def m1(j, mn):
                        lm = j * 16 + iota < nb
                        d2 = bd2[pl.ds(j * 16, 16)]
                        return jnp.minimum(mn, jnp.where(lm, d2, inf16))
                    mn = jnp.min(lax.fori_loop(0, nbw, m1, inf16))

                    def m2(j, mi):
                        lm = j * 16 + iota < nb
                        d2 = bd2[pl.ds(j * 16, 16)]
                        idxv = bidx[pl.ds(j * 16, 16)]
                        return jnp.minimum(
                            mi, jnp.where(lm & (d2 == mn), idxv, big))
                    mni = jnp.min(lax.fori_loop(0, nbw, m2, big))

                    def m3(j, _3):
                        lane = j * 16 + iota
                        lm = lane < nb
                        d2 = bd2[pl.ds(j * 16, 16)]
                        idxv = bidx[pl.ds(j * 16, 16)]
                        wm = lm & (d2 == mn) & (idxv == mni)
                        plsc.store_scatter(bd2, [lane], inf16, mask=wm)
                        return 0
                    lax.fori_loop(0, nbw, m3, 0)

                    plsc.store_scatter(
                        oidx, [jnp.full((16,), obase + cbelow + t, jnp.int32)],
                        jnp.full((16,), mni, jnp.int32), mask=iota == 0)
                    return 0

                lax.fori_loop(0, kprime, extract, 0)

            def relk(kk, _2):
                lane16 = kk * 16 + iota
                sel = plsc.load_gather(oidx, [obase + lane16])
                gx = plsc.load_gather(pxv, [sel]) - qx
                gy = plsc.load_gather(pyv, [sel]) - qy
                gz = plsc.load_gather(pzv, [sel]) - qz
                plsc.store_scatter(relb, [lane16, jnp.zeros((16,), jnp.int32)], gx)
                plsc.store_scatter(relb, [lane16, ones], gy)
                plsc.store_scatter(relb, [lane16, ones + ones], gz)
                for cc in range(3, 8):
                    plsc.store_scatter(
                        relb, [lane16, jnp.full((16,), cc, jnp.int32)], zerosf)
                return 0

            lax.fori_loop(0, _K // 16, relk, 0)
            pltpu.sync_copy(relb, rel_hbm.at[grow])
            return 0

        lax.fori_loop(0, rpw, row_fn, 0)
        pltpu.sync_copy(oidx, nbr_hbm.at[wid])
        pltpu.sync_copy(oval, val_hbm.at[wid])

    return k(pxp, pyp, pzp, qxp, qyp, qzp)


def _sc_gather(x, idx_flat):
    n, d = x.shape
    b = idx_flat.shape[0]
    nw = 32
    bpw = b // nw
    nchunk = bpw // 128
    mesh = plsc.VectorSubcoreMesh(core_axis_name="c", subcore_axis_name="s")

    @functools.partial(
        pl.kernel,
        mesh=mesh,
        out_type=jax.ShapeDtypeStruct((b, d), jnp.float32),
        scratch_types=[
            pltpu.VMEM((nchunk, 128), jnp.int32),
            pltpu.VMEM((128, d), jnp.float32),
            pltpu.VMEM((128, d), jnp.float32),
            pltpu.SemaphoreType.DMA,
            pltpu.SemaphoreType.DMA,
        ],
    )
    def k(x_hbm, idx_hbm, out_hbm, idx_v, buf0, buf1, sem0, sem1):
        wid = lax.axis_index("s") * 2 + lax.axis_index("c")
        base = wid * bpw
        pltpu.sync_copy(idx_hbm.at[wid], idx_v)
        pltpu.async_copy(x_hbm.at[idx_v.at[0]], buf0, sem0)

        def body(j, _):
            c = j * 2
            pltpu.async_copy(x_hbm.at[idx_v.at[c + 1]], buf1, sem1)
            pltpu.make_async_copy(x_hbm.at[idx_v.at[c]], buf0, sem0).wait()
            pltpu.sync_copy(buf0, out_hbm.at[pl.ds(base + c * 128, 128)])

            @pl.when(c + 2 < nchunk)
            def _():
                pltpu.async_copy(x_hbm.at[idx_v.at[c + 2]], buf0, sem0)

            pltpu.make_async_copy(x_hbm.at[idx_v.at[c + 1]], buf1, sem1).wait()
            pltpu.sync_copy(buf1, out_hbm.at[pl.ds(base + (c + 1) * 128, 128)])
            return 0

        lax.fori_loop(0, nchunk // 2, body, 0)

    return k(x, idx_flat.reshape(nw, nchunk, 128))


def _xw_body(x_ref, w_ref, out_ref):
    out_ref[...] = jnp.dot(x_ref[...], w_ref[...],
                           preferred_element_type=jnp.float32)


def _xw(x, w1x):
    n, f = x.shape
    blk = 1000
    return pl.pallas_call(
        _xw_body,
        grid=(n // blk,),
        in_specs=[
            pl.BlockSpec((blk, f), lambda i: (i, 0)),
            pl.BlockSpec((f, 128), lambda i: (0, 0)),
        ],
        out_specs=pl.BlockSpec((blk, 128), lambda i: (i, 0)),
        out_shape=jax.ShapeDtypeStruct((n, 128), jnp.float32),
    )(x, w1x)


def _mlp_body(blkc, xj_ref, rel_ref, vf_ref, w1r_ref, b1_ref,
              w2_ref, b2_ref, w3_ref, b3_ref, out_ref):
    h1 = jnp.maximum(
        xj_ref[...]
        + jnp.dot(rel_ref[...], w1r_ref[...],
                  preferred_element_type=jnp.float32)
        + b1_ref[...], 0.0)
    h2 = jnp.maximum(
        jnp.dot(h1, w2_ref[...], preferred_element_type=jnp.float32)
        + b2_ref[...], 0.0)
    h3 = jnp.maximum(
        jnp.dot(h2, w3_ref[...], preferred_element_type=jnp.float32)
        + b3_ref[...], 0.0)
    h3r = h3.reshape(blkc, _K, h3.shape[-1])
    v = vf_ref[...]
    h3m = jnp.where(v[:, :, None] > 0, h3r, -1.0)
    r = jnp.max(h3m, axis=1)
    out_ref[...] = jnp.where(r < 0, 0.0, r)


def _mlp(xj, rel8, validf, W1r8, b1, W2, b2, W3, b3, mp):
    blkc = 64
    grid = mp // blkc
    h3w = W3.shape[1]
    f = xj.shape[1]
    return pl.pallas_call(
        functools.partial(_mlp_body, blkc),
        grid=(grid,),
        in_specs=[
            pl.BlockSpec((blkc * _K, f), lambda i: (i, 0)),
            pl.BlockSpec((blkc * _K, 8), lambda i: (i, 0)),
            pl.BlockSpec((blkc, _K), lambda i: (i, 0)),
            pl.BlockSpec((8, 128), lambda i: (0, 0)),
            pl.BlockSpec((1, 128), lambda i: (0, 0)),
            pl.BlockSpec((128, 128), lambda i: (0, 0)),
            pl.BlockSpec((1, 128), lambda i: (0, 0)),
            pl.BlockSpec((128, 256), lambda i: (0, 0)),
            pl.BlockSpec((1, 256), lambda i: (0, 0)),
        ],
        out_specs=pl.BlockSpec((blkc, h3w), lambda i: (i, 0)),
        out_shape=jax.ShapeDtypeStruct((mp, h3w), jnp.float32),
    )(xj, rel8, validf, W1r8, b1, W2, b2, W3, b3)


def kernel(x, pos, batch, W1, b1, W2, b2, W3, b3):
    n, f = x.shape
    m = int(round(n * 0.25))
    rows = (n + 1023) // 1024 * 8
    idx, pos_q = _fps(pos, n, m, rows)

    mp = 2560
    npad = rows * 128
    pad = jnp.float32(1e3)
    pxp = jnp.full((npad,), pad, jnp.float32).at[:n].set(pos[:, 0])
    pyp = jnp.full((npad,), pad, jnp.float32).at[:n].set(pos[:, 1])
    pzp = jnp.full((npad,), pad, jnp.float32).at[:n].set(pos[:, 2])
    qxp = jnp.full((mp,), pad, jnp.float32).at[:m].set(pos_q[:, 0])
    qyp = jnp.full((mp,), pad, jnp.float32).at[:m].set(pos_q[:, 1])
    qzp = jnp.full((mp,), pad, jnp.float32).at[:m].set(pos_q[:, 2])
    nbr32, val32, rel = _sc_select(pxp, pyp, pzp, qxp, qyp, qzp, npad, mp)

    u = _xw(x, W1[:f])
    nbr_g = nbr32.reshape(32, mp // 32, _K).transpose(1, 0, 2)
    uj = _sc_gather(u, nbr_g.reshape(-1))
    rel8 = rel.reshape(mp * _K, 8)
    validf = val32.reshape(32, mp // 32, _K).transpose(1, 0, 2).reshape(mp, _K)
    w1r8 = jnp.zeros((8, 128), jnp.float32).at[:3, :].set(W1[f:f + 3])

    out = _mlp(uj, rel8, validf, w1r8, b1[None, :], W2, b2[None, :],
               W3, b3[None, :], mp)
    return (out[:m], pos_q, batch[idx])
